# baseline (speedup 1.0000x reference)
import functools

import jax
import jax.numpy as jnp
from jax import lax
from jax.experimental import pallas as pl
from jax.experimental.pallas import tpu as pltpu
from jax.experimental.pallas import tpu_sc as plsc

D_MODEL = 64
ROW = 128
K = 4


@functools.lru_cache(maxsize=None)
def _make_lookup(n_rows: int, d_model: int):
    info = plsc.get_sparse_core_info()
    nc, ns = info.num_cores, info.num_subcores
    nw = nc * ns
    rows_per_w = n_rows // nw
    n_chunks = rows_per_w // K
    mesh = plsc.VectorSubcoreMesh(core_axis_name="c", subcore_axis_name="s")

    @functools.partial(
        pl.kernel,
        out_type=jax.ShapeDtypeStruct((n_rows, ROW, d_model), jnp.float32),
        mesh=mesh,
        scratch_types=[
            pltpu.VMEM((K, ROW), jnp.int32),
            pltpu.VMEM((K, ROW, d_model), jnp.float32),
            pltpu.SemaphoreType.DMA,
        ],
        compiler_params=pltpu.CompilerParams(use_tc_tiling_on_sc=False),
    )
    def lookup(idx_hbm, table_hbm, out_hbm, idx_v, rows_v, sem):
        wid = lax.axis_index("s") * nc + lax.axis_index("c")
        base = wid * rows_per_w

        def body(i, carry):
            row = base + i * K
            pltpu.sync_copy(idx_hbm.at[pl.ds(row, K)], idx_v)
            copies = [
                pltpu.async_copy(table_hbm.at[idx_v.at[j]], rows_v.at[j], sem)
                for j in range(K)
            ]
            for c in copies:
                c.wait()
            pltpu.sync_copy(rows_v, out_hbm.at[pl.ds(row, K)])
            return carry

        lax.fori_loop(0, n_chunks, body, 0)

    return lookup


def kernel(x, table):
    b0, b1 = x.shape
    n = b0 * b1
    idx = x.reshape(n // ROW, ROW).astype(jnp.int32)
    out = _make_lookup(n // ROW, table.shape[1])(idx, table)
    return out.reshape(b0, b1, table.shape[1])


# speedup vs baseline: 1.0452x; 1.0452x over previous
import functools

import jax
import jax.numpy as jnp
from jax import lax
from jax.experimental import pallas as pl
from jax.experimental.pallas import tpu as pltpu
from jax.experimental.pallas import tpu_sc as plsc

ROW = 128
NBUF = 10
LOOKAHEAD = 5


@functools.lru_cache(maxsize=None)
def _make_lookup(n_rows: int, d_model: int):
    info = plsc.get_sparse_core_info()
    nc, ns = info.num_cores, info.num_subcores
    nw = nc * ns
    rows_per_w = n_rows // nw
    n = rows_per_w
    assert n % NBUF == 0
    mesh = plsc.VectorSubcoreMesh(core_axis_name="c", subcore_axis_name="s")

    @functools.partial(
        pl.kernel,
        out_type=jax.ShapeDtypeStruct((n_rows, ROW, d_model), jnp.float32),
        mesh=mesh,
        scratch_types=[
            pltpu.VMEM((rows_per_w, ROW), jnp.int32),
            pltpu.VMEM((NBUF, ROW, d_model), jnp.float32),
            [pltpu.SemaphoreType.DMA] * NBUF,
            [pltpu.SemaphoreType.DMA] * NBUF,
        ],
        compiler_params=pltpu.CompilerParams(use_tc_tiling_on_sc=False),
    )
    def lookup(idx_hbm, table_hbm, out_hbm, idx_v, rows_v, gsems, ssems):
        wid = lax.axis_index("s") * nc + lax.axis_index("c")
        base = wid * rows_per_w
        pltpu.sync_copy(idx_hbm.at[pl.ds(base, rows_per_w)], idx_v)

        def fire_gather(c, b):
            pltpu.async_copy(table_hbm.at[idx_v.at[c]], rows_v.at[b], gsems[b])

        def wait_gather(c, b):
            pltpu.make_async_copy(
                table_hbm.at[idx_v.at[c]], rows_v.at[b], gsems[b]
            ).wait()

        def fire_store(c, b):
            pltpu.async_copy(rows_v.at[b], out_hbm.at[base + c], ssems[b])

        def wait_store(c, b):
            pltpu.make_async_copy(
                rows_v.at[b], out_hbm.at[base + c], ssems[b]
            ).wait()

        for b in range(NBUF):
            fire_gather(b, b)

        def group(t, carry):
            for b in range(NBUF):
                g = t * NBUF + b
                wait_gather(g, b)
                fire_store(g, b)
                h = g + LOOKAHEAD
                hb = (b + LOOKAHEAD) % NBUF

                @pl.when(jnp.logical_and(h >= NBUF, h < n))
                def _():
                    wait_store(h - NBUF, hb)
                    fire_gather(h, hb)

            return carry

        lax.fori_loop(0, n // NBUF, group, 0)

        for b in range(NBUF):
            c = n - NBUF + b
            wait_store(c, b)

    return lookup


def kernel(x, table):
    b0, b1 = x.shape
    num = b0 * b1
    idx = x.reshape(num // ROW, ROW).astype(jnp.int32)
    out = _make_lookup(num // ROW, table.shape[1])(idx, table)
    return out.reshape(b0, b1, table.shape[1])
